# Initial kernel scaffold; baseline (speedup 1.0000x reference)
import jax
import jax.numpy as jnp
from jax.experimental import pallas as pl


def kernel(x, params):
    raise NotImplementedError("write your pallas kernel here")


# speedup vs baseline: 1.0357x; 1.0357x over previous
import functools

import jax
import jax.numpy as jnp
from jax.experimental import pallas as pl


def _conv2d(x, w, b, stride, pad):
    out = jax.lax.conv_general_dilated(
        x, w, (stride, stride), [(pad, pad), (pad, pad)],
        dimension_numbers=('NCHW', 'OIHW', 'NCHW'))
    if b is not None:
        out = out + b[None, :, None, None]
    return out


def _conv_transpose2d(x, w, b, stride, pad):
    w_t = jnp.transpose(w[:, :, ::-1, ::-1], (1, 0, 2, 3))
    k = w.shape[2]
    p = k - 1 - pad
    out = jax.lax.conv_general_dilated(
        x, w_t, (1, 1), [(p, p), (p, p)], lhs_dilation=(stride, stride),
        dimension_numbers=('NCHW', 'OIHW', 'NCHW'))
    if b is not None:
        out = out + b[None, :, None, None]
    return out


def _residual(x, wa, wb):
    h = jax.nn.relu(_conv2d(x, wa, None, 1, 1))
    return jax.nn.relu(_conv2d(h, wb, None, 1, 0))


_R = 512


def _vq_block_kernel(flat_ref, emb_ref, q_ref, minsum_ref, counts_ref, *,
                     n_valid):
    i = pl.program_id(0)

    @pl.when(i == 0)
    def _init():
        minsum_ref[...] = jnp.zeros_like(minsum_ref)
        counts_ref[...] = jnp.zeros_like(counts_ref)

    f = flat_ref[...]
    e = emb_ref[...]
    rn = jnp.sum(f * f, axis=1, keepdims=True)
    en = jnp.sum(e * e, axis=1)
    mm = jax.lax.dot_general(f, e, (((1,), (1,)), ((), ())),
                             preferred_element_type=jnp.float32)
    d = rn + en[None, :] - 2.0 * mm
    minval = jnp.min(d, axis=1, keepdims=True)
    cidx = jax.lax.broadcasted_iota(jnp.int32, d.shape, 1)
    idx = jnp.min(jnp.where(d == minval, cidx, d.shape[1]), axis=1,
                  keepdims=True)
    onehot = (cidx == idx).astype(jnp.float32)
    q_ref[...] = jax.lax.dot_general(onehot, e, (((1,), (0,)), ((), ())),
                                     preferred_element_type=jnp.float32)
    rows = i * _R + jax.lax.broadcasted_iota(jnp.int32, (_R, 1), 0)
    vmask = (rows < n_valid).astype(jnp.float32)
    minsum_ref[...] += jnp.sum(minval * vmask).reshape(1, 1)
    counts_ref[...] += jnp.sum(onehot * vmask, axis=0)[None, :]


def _vq_quantize(flat, emb):
    n, dim = flat.shape
    k = emb.shape[0]
    npad = (-n) % _R
    flat_p = jnp.pad(flat, ((0, npad), (0, 0)))
    n_p = n + npad
    q, minsum, counts = pl.pallas_call(
        functools.partial(_vq_block_kernel, n_valid=n),
        grid=(n_p // _R,),
        in_specs=[pl.BlockSpec((_R, dim), lambda i: (i, 0)),
                  pl.BlockSpec((k, dim), lambda i: (0, 0))],
        out_specs=[pl.BlockSpec((_R, dim), lambda i: (i, 0)),
                   pl.BlockSpec((1, 1), lambda i: (0, 0)),
                   pl.BlockSpec((1, k), lambda i: (0, 0))],
        out_shape=[jax.ShapeDtypeStruct((n_p, dim), jnp.float32),
                   jax.ShapeDtypeStruct((1, 1), jnp.float32),
                   jax.ShapeDtypeStruct((1, k), jnp.float32)],
    )(flat_p, emb)
    return q[:n], minsum[0, 0], counts[0]


def _vq(z, emb):
    x = jnp.transpose(z, (0, 2, 3, 1))
    shp = x.shape
    flat = x.reshape(-1, emb.shape[1])
    n = flat.shape[0]
    q, minsum, counts = _vq_quantize(flat, emb)
    loss = 1.25 * (minsum / (n * emb.shape[1]))
    qr = q.reshape(shp)
    q_st = x + (qr - x)
    avg = counts / n
    perp = jnp.exp(-jnp.sum(avg * jnp.log(avg + 1e-10)))
    return loss, jnp.transpose(q_st, (0, 3, 1, 2)), perp


def kernel(x, params):
    p = params
    h = jax.nn.relu(_conv2d(x, p['eb_c1_w'], p['eb_c1_b'], 2, 1))
    h = jax.nn.relu(_conv2d(h, p['eb_c2_w'], p['eb_c2_b'], 2, 1))
    h = jax.nn.relu(_conv2d(h, p['eb_c3_w'], p['eb_c3_b'], 1, 1))
    h = _residual(h, p['eb_r1a_w'], p['eb_r1b_w'])
    z_bottom = _residual(h, p['eb_r2a_w'], p['eb_r2b_w'])
    h = jax.nn.relu(_conv2d(z_bottom, p['et_c1_w'], p['et_c1_b'], 2, 1))
    h = jax.nn.relu(_conv2d(h, p['et_c2_w'], p['et_c2_b'], 1, 1))
    h = _residual(h, p['et_r1a_w'], p['et_r1b_w'])
    z_top = _residual(h, p['et_r2a_w'], p['et_r2b_w'])
    loss_top, q_top, pt = _vq(_conv2d(z_top, p['pvt_w'], p['pvt_b'], 1, 0),
                              p['emb_top'])
    h = _conv2d(q_top, p['dt_c1_w'], p['dt_c1_b'], 1, 1)
    h = _residual(h, p['dt_r1a_w'], p['dt_r1b_w'])
    h = _residual(h, p['dt_r2a_w'], p['dt_r2b_w'])
    rec_top = _conv_transpose2d(h, p['dt_t1_w'], p['dt_t1_b'], 2, 1)
    zb = jnp.concatenate([rec_top, z_bottom], axis=1)
    loss_bottom, q_bot, pb = _vq(_conv2d(zb, p['pvb_w'], p['pvb_b'], 1, 0),
                                 p['emb_bot'])
    up = _conv_transpose2d(q_top, p['up_w'], p['up_b'], 2, 1)
    quantized = jnp.concatenate([up, q_bot], axis=1)
    h = _conv2d(quantized, p['db_c1_w'], p['db_c1_b'], 1, 1)
    h = _residual(h, p['db_r1a_w'], p['db_r1b_w'])
    h = _residual(h, p['db_r2a_w'], p['db_r2b_w'])
    h = jax.nn.relu(_conv_transpose2d(h, p['db_t1_w'], p['db_t1_b'], 2, 1))
    x_rec = _conv_transpose2d(h, p['db_t2_w'], p['db_t2_b'], 2, 1)
    return loss_top + loss_bottom, x_rec, pt + pb, quantized
